# baseline (speedup 1.0000x reference)
import functools

import jax
import jax.numpy as jnp
from jax import lax
from jax.experimental import pallas as pl
from jax.experimental.pallas import tpu as pltpu
from jax.experimental.pallas import tpu_sc as plsc

_N = 10000
_E = 320000
_H = 128

_NC = 2
_NS = 16
_NW = _NC * _NS
_B = 80
_EPT = _E // _NW
_RSTRIDE = 624
_RCOPY = 640
_DEGW = 16

_f32 = jnp.float32


_B2 = 128
_CPT = 80
_PH = 5
_PCH = _CPT // _PH
_EPT2 = _B2 * _CPT
_EPAD = _NW * _EPT2
_NTRASH = 2048
_NP = _N + _NTRASH


def _seg_body(u_hbm, src_hbm, dst_hbm, z_hbm, out_hbm,
              idx_s, idx_d, rows, acc, sem):
    c = lax.axis_index("c")
    s = lax.axis_index("s")
    wid = c * _NS + s
    r0 = s * _RSTRIDE
    pltpu.sync_copy(z_hbm.at[pl.ds(r0, _RCOPY)], acc.at[pl.ds(r0, _RCOPY)])
    plsc.subcore_barrier()
    base = wid * _EPT2

    def chunk(k, carry):
        off = base + k * _B2
        pltpu.sync_copy(src_hbm.at[pl.ds(off, _B2)], idx_s)
        pltpu.sync_copy(dst_hbm.at[pl.ds(off, _B2)], idx_d)
        pltpu.async_copy(u_hbm.at[idx_s], rows, sem).wait()
        pltpu.sync_copy(rows, acc.at[idx_d], add=True)
        return carry

    lax.fori_loop(0, _CPT, chunk, 0)
    plsc.subcore_barrier()
    pltpu.sync_copy(acc.at[pl.ds(r0, _RCOPY)],
                    out_hbm.at[pl.ds(c * _N + r0, _RCOPY)])


_sc_seg = pl.kernel(
    _seg_body,
    out_type=jax.ShapeDtypeStruct((2 * _N, _H), _f32),
    mesh=plsc.VectorSubcoreMesh(core_axis_name="c", subcore_axis_name="s"),
    scratch_types=[
        pltpu.VMEM((_B2,), jnp.int32),
        pltpu.VMEM((_B2,), jnp.int32),
        pltpu.VMEM((_B2, _H), _f32),
        pltpu.VMEM_SHARED((_NP, _H), _f32),
        pltpu.SemaphoreType.DMA,
    ],
)


def _deg_body(dst_hbm, z_hbm, ones_hbm, out_hbm, idx_d, ones_v, bounce, acc):
    c = lax.axis_index("c")
    s = lax.axis_index("s")
    wid = c * _NS + s
    r0 = s * _RSTRIDE
    pltpu.sync_copy(z_hbm.at[pl.ds(r0, _RCOPY)], bounce)
    pltpu.sync_copy(bounce, acc.at[pl.ds(r0, _RCOPY)])
    pltpu.sync_copy(ones_hbm, ones_v)
    plsc.subcore_barrier()
    base = wid * _EPT

    def chunk(k, carry):
        off = base + k * _B
        pltpu.sync_copy(dst_hbm.at[pl.ds(off, _B)], idx_d)
        pltpu.sync_copy(ones_v, acc.at[idx_d], add=True)
        return carry

    lax.fori_loop(0, _EPT // _B, chunk, 0)
    plsc.subcore_barrier()
    pltpu.sync_copy(acc.at[pl.ds(r0, _RCOPY)], bounce)
    pltpu.sync_copy(bounce, out_hbm.at[pl.ds(c * _N + r0, _RCOPY)])


_sc_deg = pl.kernel(
    _deg_body,
    out_type=jax.ShapeDtypeStruct((2 * _N,), _f32),
    mesh=plsc.VectorSubcoreMesh(core_axis_name="c", subcore_axis_name="s"),
    scratch_types=[
        pltpu.VMEM((_B,), jnp.int32),
        pltpu.VMEM((_B,), _f32),
        pltpu.VMEM((_RCOPY,), _f32),
        pltpu.VMEM_SHARED((_N,), _f32),
    ],
)


_R = 1000
_G = _N // _R


def _mm_body(x_ref, w_ref, o_ref):
    o_ref[...] = jnp.dot(x_ref[...], w_ref[...],
                         preferred_element_type=_f32)


_tc_matmul = pl.pallas_call(
    _mm_body,
    grid=(_G,),
    in_specs=[
        pl.BlockSpec((_R, _H), lambda i: (i, 0)),
        pl.BlockSpec((_H, _H), lambda i: (0, 0)),
    ],
    out_specs=pl.BlockSpec((_R, _H), lambda i: (i, 0)),
    out_shape=jax.ShapeDtypeStruct((_N, _H), _f32),
)


def _scale_body(dga_ref, dgb_ref, p_ref, u_ref, dinv_ref, selfc_ref):
    deg = dga_ref[...] + dgb_ref[...] + 1.0
    dinv = lax.rsqrt(deg)
    selfc = 1.0 / deg
    u_ref[...] = dinv * p_ref[...]
    dinv_ref[...] = dinv
    selfc_ref[...] = selfc


_tc_scale = pl.pallas_call(
    _scale_body,
    grid=(_G,),
    in_specs=[
        pl.BlockSpec((_R, 1), lambda i: (i, 0)),
        pl.BlockSpec((_R, 1), lambda i: (i + _G, 0)),
        pl.BlockSpec((_R, _H), lambda i: (i, 0)),
    ],
    out_specs=[
        pl.BlockSpec((_R, _H), lambda i: (i, 0)),
        pl.BlockSpec((_R, 1), lambda i: (i, 0)),
        pl.BlockSpec((_R, 1), lambda i: (i, 0)),
    ],
    out_shape=[
        jax.ShapeDtypeStruct((_N, _H), _f32),
        jax.ShapeDtypeStruct((_N, 1), _f32),
        jax.ShapeDtypeStruct((_N, 1), _f32),
    ],
)


def _cmb_mm_body(sa_ref, sb_ref, p_ref, dinv_ref, selfc_ref, b_ref, w_ref,
                 pn_ref, un_ref):
    h = (dinv_ref[...] * (sa_ref[...] + sb_ref[...])
         + selfc_ref[...] * p_ref[...] + b_ref[...])
    pn = jnp.dot(h, w_ref[...], preferred_element_type=_f32)
    pn_ref[...] = pn
    un_ref[...] = dinv_ref[...] * pn


_tc_combine_mm = pl.pallas_call(
    _cmb_mm_body,
    grid=(_G,),
    in_specs=[
        pl.BlockSpec((_R, _H), lambda i: (i, 0)),
        pl.BlockSpec((_R, _H), lambda i: (i + _G, 0)),
        pl.BlockSpec((_R, _H), lambda i: (i, 0)),
        pl.BlockSpec((_R, 1), lambda i: (i, 0)),
        pl.BlockSpec((_R, 1), lambda i: (i, 0)),
        pl.BlockSpec((1, _H), lambda i: (0, 0)),
        pl.BlockSpec((_H, _H), lambda i: (0, 0)),
    ],
    out_specs=[
        pl.BlockSpec((_R, _H), lambda i: (i, 0)),
        pl.BlockSpec((_R, _H), lambda i: (i, 0)),
    ],
    out_shape=[
        jax.ShapeDtypeStruct((_N, _H), _f32),
        jax.ShapeDtypeStruct((_N, _H), _f32),
    ],
)


def _cmb_body(sa_ref, sb_ref, p_ref, dinv_ref, selfc_ref, b_ref, h_ref):
    h_ref[...] = (dinv_ref[...] * (sa_ref[...] + sb_ref[...])
                  + selfc_ref[...] * p_ref[...] + b_ref[...])


_tc_combine = pl.pallas_call(
    _cmb_body,
    grid=(_G,),
    in_specs=[
        pl.BlockSpec((_R, _H), lambda i: (i, 0)),
        pl.BlockSpec((_R, _H), lambda i: (i + _G, 0)),
        pl.BlockSpec((_R, _H), lambda i: (i, 0)),
        pl.BlockSpec((_R, 1), lambda i: (i, 0)),
        pl.BlockSpec((_R, 1), lambda i: (i, 0)),
        pl.BlockSpec((1, _H), lambda i: (0, 0)),
    ],
    out_specs=pl.BlockSpec((_R, _H), lambda i: (i, 0)),
    out_shape=jax.ShapeDtypeStruct((_N, _H), _f32),
)


def _elu(v):
    return jnp.where(v > 0, v, jnp.exp(v) - 1.0)


def _gin_body(h3_ref, sa_ref, sb_ref,
              wp1_ref, bp1_ref, wp2_ref, bp2_ref,
              wv1_ref, bv1_ref, wv2_ref, bv2_ref,
              proba_ref, value_ref, accv_ref):
    i = pl.program_id(0)
    g = h3_ref[...] + sa_ref[...] + sb_ref[...]
    tp = _elu(jnp.dot(g, wp1_ref[...], preferred_element_type=_f32)
              + bp1_ref[...])
    proba_ref[...] = (jnp.dot(tp, wp2_ref[...], preferred_element_type=_f32)
                      + bp2_ref[...])
    tv = _elu(jnp.dot(g, wv1_ref[...], preferred_element_type=_f32)
              + bv1_ref[...])
    sv = jnp.sum(tv, axis=0, keepdims=True)

    @pl.when(i == 0)
    def _():
        accv_ref[...] = sv

    @pl.when(i > 0)
    def _():
        accv_ref[...] += sv

    @pl.when(i == pl.num_programs(0) - 1)
    def _():
        value_ref[...] = (jnp.dot(accv_ref[...] / _N, wv2_ref[...],
                                  preferred_element_type=_f32)
                          + bv2_ref[...])


_tc_gin = pl.pallas_call(
    _gin_body,
    grid=(_G,),
    in_specs=[
        pl.BlockSpec((_R, _H), lambda i: (i, 0)),
        pl.BlockSpec((_R, _H), lambda i: (i, 0)),
        pl.BlockSpec((_R, _H), lambda i: (i + _G, 0)),
        pl.BlockSpec((_H, _H), lambda i: (0, 0)),
        pl.BlockSpec((1, _H), lambda i: (0, 0)),
        pl.BlockSpec((_H, 1), lambda i: (0, 0)),
        pl.BlockSpec((1, 1), lambda i: (0, 0)),
        pl.BlockSpec((_H, _H), lambda i: (0, 0)),
        pl.BlockSpec((1, _H), lambda i: (0, 0)),
        pl.BlockSpec((_H, 1), lambda i: (0, 0)),
        pl.BlockSpec((1, 1), lambda i: (0, 0)),
    ],
    out_specs=[
        pl.BlockSpec((_R, 1), lambda i: (i, 0)),
        pl.BlockSpec((1, 1), lambda i: (0, 0)),
    ],
    out_shape=[
        jax.ShapeDtypeStruct((_N, 1), _f32),
        jax.ShapeDtypeStruct((1, 1), _f32),
    ],
    scratch_shapes=[pltpu.VMEM((1, _H), _f32)],
)


def kernel(x, edge_index, W1, b1, W2, b2, W3, b3,
           Wv1, bv1, Wv2, bv2, Wp1, bp1, Wp2, bp2):
    src = edge_index[0]
    dst = edge_index[1]
    npad = _EPAD - _E
    src2 = jnp.concatenate([src, jnp.zeros((npad,), jnp.int32)])
    trash = _N + (jnp.arange(npad, dtype=jnp.int32) % _NTRASH)
    dst2 = jnp.concatenate([dst, trash])
    z128 = jnp.zeros((_N, _H), _f32)
    z1 = jnp.zeros((_N,), _f32)
    ones1 = jnp.ones((_B,), _f32)
    b1r = b1.reshape(1, _H)
    b2r = b2.reshape(1, _H)
    b3r = b3.reshape(1, _H)

    dg = _sc_deg(dst, z1, ones1).reshape(2 * _N, 1)
    p1 = _tc_matmul(x, W1)
    u1, dinv, selfc = _tc_scale(dg, dg, p1)
    s1 = _sc_seg(u1, src2, dst2, z128)
    p2, u2 = _tc_combine_mm(s1, s1, p1, dinv, selfc, b1r, W2)
    s2 = _sc_seg(u2, src2, dst2, z128)
    p3, u3 = _tc_combine_mm(s2, s2, p2, dinv, selfc, b2r, W3)
    s3 = _sc_seg(u3, src2, dst2, z128)
    h3 = _tc_combine(s3, s3, p3, dinv, selfc, b3r)
    s4 = _sc_seg(h3, src2, dst2, z128)
    proba, value = _tc_gin(h3, s4, s4, Wp1, bp1.reshape(1, _H), Wp2,
                           bp2.reshape(1, 1), Wv1, bv1.reshape(1, _H), Wv2,
                           bv2.reshape(1, 1))
    return proba, value


# speedup vs baseline: 1.1555x; 1.1555x over previous
import functools

import jax
import jax.numpy as jnp
from jax import lax
from jax.experimental import pallas as pl
from jax.experimental.pallas import tpu as pltpu
from jax.experimental.pallas import tpu_sc as plsc

_N = 10000
_E = 320000
_H = 128

_NC = 2
_NS = 16
_NW = _NC * _NS
_B = 80
_EPT = _E // _NW
_RSTRIDE = 624
_RCOPY = 640
_DEGW = 16

_f32 = jnp.float32


_B2 = 40
_CPT = 250
_PH = 5
_PCH = _CPT // _PH
_EPT2 = _B2 * _CPT
_EPAD = _NW * _EPT2
_NTRASH = 2048
_NP = _N + _NTRASH


def _seg_body(u_hbm, src_hbm, dst_hbm, z_hbm, out_hbm,
              idx_s, idx_d, rows, acc, sem):
    c = lax.axis_index("c")
    s = lax.axis_index("s")
    wid = c * _NS + s
    r0 = s * _RSTRIDE
    pltpu.sync_copy(z_hbm.at[pl.ds(r0, _RCOPY)], acc.at[pl.ds(r0, _RCOPY)])
    plsc.subcore_barrier()
    base = wid * _EPT2

    def chunk(k, carry):
        off = base + k * _B2
        pltpu.sync_copy(src_hbm.at[pl.ds(off, _B2)], idx_s)
        pltpu.sync_copy(dst_hbm.at[pl.ds(off, _B2)], idx_d)
        pltpu.async_copy(u_hbm.at[idx_s], rows, sem).wait()
        pltpu.sync_copy(rows, acc.at[idx_d], add=True)
        return carry

    lax.fori_loop(0, _CPT, chunk, 0)
    plsc.subcore_barrier()
    pltpu.sync_copy(acc.at[pl.ds(r0, _RCOPY)],
                    out_hbm.at[pl.ds(c * _N + r0, _RCOPY)])


_sc_seg = pl.kernel(
    _seg_body,
    out_type=jax.ShapeDtypeStruct((2 * _N, _H), _f32),
    mesh=plsc.VectorSubcoreMesh(core_axis_name="c", subcore_axis_name="s"),
    scratch_types=[
        pltpu.VMEM((_B2,), jnp.int32),
        pltpu.VMEM((_B2,), jnp.int32),
        pltpu.VMEM((_B2, _H), _f32),
        pltpu.VMEM_SHARED((_NP, _H), _f32),
        pltpu.SemaphoreType.DMA,
    ],
)


def _deg_body(dst_hbm, z_hbm, ones_hbm, out_hbm, idx_d, ones_v, bounce, acc):
    c = lax.axis_index("c")
    s = lax.axis_index("s")
    wid = c * _NS + s
    r0 = s * _RSTRIDE
    pltpu.sync_copy(z_hbm.at[pl.ds(r0, _RCOPY)], bounce)
    pltpu.sync_copy(bounce, acc.at[pl.ds(r0, _RCOPY)])
    pltpu.sync_copy(ones_hbm, ones_v)
    plsc.subcore_barrier()
    base = wid * _EPT

    def chunk(k, carry):
        off = base + k * _B
        pltpu.sync_copy(dst_hbm.at[pl.ds(off, _B)], idx_d)
        pltpu.sync_copy(ones_v, acc.at[idx_d], add=True)
        return carry

    lax.fori_loop(0, _EPT // _B, chunk, 0)
    plsc.subcore_barrier()
    pltpu.sync_copy(acc.at[pl.ds(r0, _RCOPY)], bounce)
    pltpu.sync_copy(bounce, out_hbm.at[pl.ds(c * _N + r0, _RCOPY)])


_sc_deg = pl.kernel(
    _deg_body,
    out_type=jax.ShapeDtypeStruct((2 * _N,), _f32),
    mesh=plsc.VectorSubcoreMesh(core_axis_name="c", subcore_axis_name="s"),
    scratch_types=[
        pltpu.VMEM((_B,), jnp.int32),
        pltpu.VMEM((_B,), _f32),
        pltpu.VMEM((_RCOPY,), _f32),
        pltpu.VMEM_SHARED((_N,), _f32),
    ],
)


_R = 1000
_G = _N // _R


def _mm_body(x_ref, w_ref, o_ref):
    o_ref[...] = jnp.dot(x_ref[...], w_ref[...],
                         preferred_element_type=_f32)


_tc_matmul = pl.pallas_call(
    _mm_body,
    grid=(_G,),
    in_specs=[
        pl.BlockSpec((_R, _H), lambda i: (i, 0)),
        pl.BlockSpec((_H, _H), lambda i: (0, 0)),
    ],
    out_specs=pl.BlockSpec((_R, _H), lambda i: (i, 0)),
    out_shape=jax.ShapeDtypeStruct((_N, _H), _f32),
)


def _scale_body(dga_ref, dgb_ref, p_ref, u_ref, dinv_ref, selfc_ref):
    deg = dga_ref[...] + dgb_ref[...] + 1.0
    dinv = lax.rsqrt(deg)
    selfc = 1.0 / deg
    u_ref[...] = dinv * p_ref[...]
    dinv_ref[...] = dinv
    selfc_ref[...] = selfc


_tc_scale = pl.pallas_call(
    _scale_body,
    grid=(_G,),
    in_specs=[
        pl.BlockSpec((_R, 1), lambda i: (i, 0)),
        pl.BlockSpec((_R, 1), lambda i: (i + _G, 0)),
        pl.BlockSpec((_R, _H), lambda i: (i, 0)),
    ],
    out_specs=[
        pl.BlockSpec((_R, _H), lambda i: (i, 0)),
        pl.BlockSpec((_R, 1), lambda i: (i, 0)),
        pl.BlockSpec((_R, 1), lambda i: (i, 0)),
    ],
    out_shape=[
        jax.ShapeDtypeStruct((_N, _H), _f32),
        jax.ShapeDtypeStruct((_N, 1), _f32),
        jax.ShapeDtypeStruct((_N, 1), _f32),
    ],
)


def _cmb_mm_body(sa_ref, sb_ref, p_ref, dinv_ref, selfc_ref, b_ref, w_ref,
                 pn_ref, un_ref):
    h = (dinv_ref[...] * (sa_ref[...] + sb_ref[...])
         + selfc_ref[...] * p_ref[...] + b_ref[...])
    pn = jnp.dot(h, w_ref[...], preferred_element_type=_f32)
    pn_ref[...] = pn
    un_ref[...] = dinv_ref[...] * pn


_tc_combine_mm = pl.pallas_call(
    _cmb_mm_body,
    grid=(_G,),
    in_specs=[
        pl.BlockSpec((_R, _H), lambda i: (i, 0)),
        pl.BlockSpec((_R, _H), lambda i: (i + _G, 0)),
        pl.BlockSpec((_R, _H), lambda i: (i, 0)),
        pl.BlockSpec((_R, 1), lambda i: (i, 0)),
        pl.BlockSpec((_R, 1), lambda i: (i, 0)),
        pl.BlockSpec((1, _H), lambda i: (0, 0)),
        pl.BlockSpec((_H, _H), lambda i: (0, 0)),
    ],
    out_specs=[
        pl.BlockSpec((_R, _H), lambda i: (i, 0)),
        pl.BlockSpec((_R, _H), lambda i: (i, 0)),
    ],
    out_shape=[
        jax.ShapeDtypeStruct((_N, _H), _f32),
        jax.ShapeDtypeStruct((_N, _H), _f32),
    ],
)


def _cmb_body(sa_ref, sb_ref, p_ref, dinv_ref, selfc_ref, b_ref, h_ref):
    h_ref[...] = (dinv_ref[...] * (sa_ref[...] + sb_ref[...])
                  + selfc_ref[...] * p_ref[...] + b_ref[...])


_tc_combine = pl.pallas_call(
    _cmb_body,
    grid=(_G,),
    in_specs=[
        pl.BlockSpec((_R, _H), lambda i: (i, 0)),
        pl.BlockSpec((_R, _H), lambda i: (i + _G, 0)),
        pl.BlockSpec((_R, _H), lambda i: (i, 0)),
        pl.BlockSpec((_R, 1), lambda i: (i, 0)),
        pl.BlockSpec((_R, 1), lambda i: (i, 0)),
        pl.BlockSpec((1, _H), lambda i: (0, 0)),
    ],
    out_specs=pl.BlockSpec((_R, _H), lambda i: (i, 0)),
    out_shape=jax.ShapeDtypeStruct((_N, _H), _f32),
)


def _elu(v):
    return jnp.where(v > 0, v, jnp.exp(v) - 1.0)


def _gin_body(h3_ref, sa_ref, sb_ref,
              wp1_ref, bp1_ref, wp2_ref, bp2_ref,
              wv1_ref, bv1_ref, wv2_ref, bv2_ref,
              proba_ref, value_ref, accv_ref):
    i = pl.program_id(0)
    g = h3_ref[...] + sa_ref[...] + sb_ref[...]
    tp = _elu(jnp.dot(g, wp1_ref[...], preferred_element_type=_f32)
              + bp1_ref[...])
    proba_ref[...] = (jnp.dot(tp, wp2_ref[...], preferred_element_type=_f32)
                      + bp2_ref[...])
    tv = _elu(jnp.dot(g, wv1_ref[...], preferred_element_type=_f32)
              + bv1_ref[...])
    sv = jnp.sum(tv, axis=0, keepdims=True)

    @pl.when(i == 0)
    def _():
        accv_ref[...] = sv

    @pl.when(i > 0)
    def _():
        accv_ref[...] += sv

    @pl.when(i == pl.num_programs(0) - 1)
    def _():
        value_ref[...] = (jnp.dot(accv_ref[...] / _N, wv2_ref[...],
                                  preferred_element_type=_f32)
                          + bv2_ref[...])


_tc_gin = pl.pallas_call(
    _gin_body,
    grid=(_G,),
    in_specs=[
        pl.BlockSpec((_R, _H), lambda i: (i, 0)),
        pl.BlockSpec((_R, _H), lambda i: (i, 0)),
        pl.BlockSpec((_R, _H), lambda i: (i + _G, 0)),
        pl.BlockSpec((_H, _H), lambda i: (0, 0)),
        pl.BlockSpec((1, _H), lambda i: (0, 0)),
        pl.BlockSpec((_H, 1), lambda i: (0, 0)),
        pl.BlockSpec((1, 1), lambda i: (0, 0)),
        pl.BlockSpec((_H, _H), lambda i: (0, 0)),
        pl.BlockSpec((1, _H), lambda i: (0, 0)),
        pl.BlockSpec((_H, 1), lambda i: (0, 0)),
        pl.BlockSpec((1, 1), lambda i: (0, 0)),
    ],
    out_specs=[
        pl.BlockSpec((_R, 1), lambda i: (i, 0)),
        pl.BlockSpec((1, 1), lambda i: (0, 0)),
    ],
    out_shape=[
        jax.ShapeDtypeStruct((_N, 1), _f32),
        jax.ShapeDtypeStruct((1, 1), _f32),
    ],
    scratch_shapes=[pltpu.VMEM((1, _H), _f32)],
)


def kernel(x, edge_index, W1, b1, W2, b2, W3, b3,
           Wv1, bv1, Wv2, bv2, Wp1, bp1, Wp2, bp2):
    src = edge_index[0]
    dst = edge_index[1]
    npad = _EPAD - _E
    src2 = jnp.concatenate([src, jnp.zeros((npad,), jnp.int32)])
    trash = _N + (jnp.arange(npad, dtype=jnp.int32) % _NTRASH)
    dst2 = jnp.concatenate([dst, trash])
    z128 = jnp.zeros((_N, _H), _f32)
    z1 = jnp.zeros((_N,), _f32)
    ones1 = jnp.ones((_B,), _f32)
    b1r = b1.reshape(1, _H)
    b2r = b2.reshape(1, _H)
    b3r = b3.reshape(1, _H)

    dg = _sc_deg(dst, z1, ones1).reshape(2 * _N, 1)
    p1 = _tc_matmul(x, W1)
    u1, dinv, selfc = _tc_scale(dg, dg, p1)
    s1 = _sc_seg(u1, src2, dst2, z128)
    p2, u2 = _tc_combine_mm(s1, s1, p1, dinv, selfc, b1r, W2)
    s2 = _sc_seg(u2, src2, dst2, z128)
    p3, u3 = _tc_combine_mm(s2, s2, p2, dinv, selfc, b2r, W3)
    s3 = _sc_seg(u3, src2, dst2, z128)
    h3 = _tc_combine(s3, s3, p3, dinv, selfc, b3r)
    s4 = _sc_seg(h3, src2, dst2, z128)
    proba, value = _tc_gin(h3, s4, s4, Wp1, bp1.reshape(1, _H), Wp2,
                           bp2.reshape(1, 1), Wv1, bv1.reshape(1, _H), Wv2,
                           bv2.reshape(1, 1))
    return proba, value


# speedup vs baseline: 1.9379x; 1.6772x over previous
import functools

import jax
import jax.numpy as jnp
from jax import lax
from jax.experimental import pallas as pl
from jax.experimental.pallas import tpu as pltpu
from jax.experimental.pallas import tpu_sc as plsc

_N = 10000
_E = 320000
_H = 128

_NC = 2
_NS = 16
_NW = _NC * _NS
_B = 80
_EPT = _E // _NW
_RSTRIDE = 624
_RCOPY = 640
_DEGW = 16

_f32 = jnp.float32


_B2 = 80
_CPT = 125
_PH = 5
_PCH = _CPT // _PH
_EPT2 = _B2 * _CPT
_EPAD = _NW * _EPT2
_NTRASH = 2048
_NP = _N + _NTRASH


def _seg_body(u_hbm, esd_hbm, z_hbm, out_hbm,
              ebuf, rows, acc, sem):
    c = lax.axis_index("c")
    s = lax.axis_index("s")
    wid = c * _NS + s
    r0 = s * _RSTRIDE
    pltpu.sync_copy(z_hbm.at[pl.ds(r0, _RCOPY)], acc.at[pl.ds(r0, _RCOPY)])
    plsc.subcore_barrier()
    base = wid * _CPT

    def chunk(k, carry):
        pltpu.sync_copy(esd_hbm.at[pl.ds((base + k) * 8, 8)], ebuf)
        pltpu.async_copy(u_hbm.at[ebuf.at[0]], rows, sem).wait()
        pltpu.sync_copy(rows, acc.at[ebuf.at[1]], add=True)
        return carry

    lax.fori_loop(0, _CPT, chunk, 0)
    plsc.subcore_barrier()
    pltpu.sync_copy(acc.at[pl.ds(r0, _RCOPY)],
                    out_hbm.at[pl.ds(c * _N + r0, _RCOPY)])


_sc_seg = pl.kernel(
    _seg_body,
    out_type=jax.ShapeDtypeStruct((2 * _N, _H), _f32),
    mesh=plsc.VectorSubcoreMesh(core_axis_name="c", subcore_axis_name="s"),
    scratch_types=[
        pltpu.VMEM((8, _B2), jnp.int32),
        pltpu.VMEM((_B2, _H), _f32),
        pltpu.VMEM_SHARED((_NP, _H), _f32),
        pltpu.SemaphoreType.DMA,
    ],
)


def _deg_body(dst_hbm, z_hbm, ones_hbm, out_hbm, idx_d, ones_v, bounce, acc):
    c = lax.axis_index("c")
    s = lax.axis_index("s")
    wid = c * _NS + s
    r0 = s * _RSTRIDE
    pltpu.sync_copy(z_hbm.at[pl.ds(r0, _RCOPY)], bounce)
    pltpu.sync_copy(bounce, acc.at[pl.ds(r0, _RCOPY)])
    pltpu.sync_copy(ones_hbm, ones_v)
    plsc.subcore_barrier()
    base = wid * _EPT

    def chunk(k, carry):
        off = base + k * _B
        pltpu.sync_copy(dst_hbm.at[pl.ds(off, _B)], idx_d)
        pltpu.sync_copy(ones_v, acc.at[idx_d], add=True)
        return carry

    lax.fori_loop(0, _EPT // _B, chunk, 0)
    plsc.subcore_barrier()
    pltpu.sync_copy(acc.at[pl.ds(r0, _RCOPY)], bounce)
    pltpu.sync_copy(bounce, out_hbm.at[pl.ds(c * _N + r0, _RCOPY)])


_sc_deg = pl.kernel(
    _deg_body,
    out_type=jax.ShapeDtypeStruct((2 * _N,), _f32),
    mesh=plsc.VectorSubcoreMesh(core_axis_name="c", subcore_axis_name="s"),
    scratch_types=[
        pltpu.VMEM((_B,), jnp.int32),
        pltpu.VMEM((_B,), _f32),
        pltpu.VMEM((_RCOPY,), _f32),
        pltpu.VMEM_SHARED((_N,), _f32),
    ],
)


_R = 1000
_G = _N // _R


def _mm_body(x_ref, w_ref, o_ref):
    o_ref[...] = jnp.dot(x_ref[...], w_ref[...],
                         preferred_element_type=_f32)


_tc_matmul = pl.pallas_call(
    _mm_body,
    grid=(_G,),
    in_specs=[
        pl.BlockSpec((_R, _H), lambda i: (i, 0)),
        pl.BlockSpec((_H, _H), lambda i: (0, 0)),
    ],
    out_specs=pl.BlockSpec((_R, _H), lambda i: (i, 0)),
    out_shape=jax.ShapeDtypeStruct((_N, _H), _f32),
)


def _scale_body(dga_ref, dgb_ref, p_ref, u_ref, dinv_ref, selfc_ref):
    deg = dga_ref[...] + dgb_ref[...] + 1.0
    dinv = lax.rsqrt(deg)
    selfc = 1.0 / deg
    u_ref[...] = dinv * p_ref[...]
    dinv_ref[...] = dinv
    selfc_ref[...] = selfc


_tc_scale = pl.pallas_call(
    _scale_body,
    grid=(_G,),
    in_specs=[
        pl.BlockSpec((_R, 1), lambda i: (i, 0)),
        pl.BlockSpec((_R, 1), lambda i: (i + _G, 0)),
        pl.BlockSpec((_R, _H), lambda i: (i, 0)),
    ],
    out_specs=[
        pl.BlockSpec((_R, _H), lambda i: (i, 0)),
        pl.BlockSpec((_R, 1), lambda i: (i, 0)),
        pl.BlockSpec((_R, 1), lambda i: (i, 0)),
    ],
    out_shape=[
        jax.ShapeDtypeStruct((_N, _H), _f32),
        jax.ShapeDtypeStruct((_N, 1), _f32),
        jax.ShapeDtypeStruct((_N, 1), _f32),
    ],
)


def _cmb_mm_body(sa_ref, sb_ref, p_ref, dinv_ref, selfc_ref, b_ref, w_ref,
                 pn_ref, un_ref):
    h = (dinv_ref[...] * (sa_ref[...] + sb_ref[...])
         + selfc_ref[...] * p_ref[...] + b_ref[...])
    pn = jnp.dot(h, w_ref[...], preferred_element_type=_f32)
    pn_ref[...] = pn
    un_ref[...] = dinv_ref[...] * pn


_tc_combine_mm = pl.pallas_call(
    _cmb_mm_body,
    grid=(_G,),
    in_specs=[
        pl.BlockSpec((_R, _H), lambda i: (i, 0)),
        pl.BlockSpec((_R, _H), lambda i: (i + _G, 0)),
        pl.BlockSpec((_R, _H), lambda i: (i, 0)),
        pl.BlockSpec((_R, 1), lambda i: (i, 0)),
        pl.BlockSpec((_R, 1), lambda i: (i, 0)),
        pl.BlockSpec((1, _H), lambda i: (0, 0)),
        pl.BlockSpec((_H, _H), lambda i: (0, 0)),
    ],
    out_specs=[
        pl.BlockSpec((_R, _H), lambda i: (i, 0)),
        pl.BlockSpec((_R, _H), lambda i: (i, 0)),
    ],
    out_shape=[
        jax.ShapeDtypeStruct((_N, _H), _f32),
        jax.ShapeDtypeStruct((_N, _H), _f32),
    ],
)


def _cmb_body(sa_ref, sb_ref, p_ref, dinv_ref, selfc_ref, b_ref, h_ref):
    h_ref[...] = (dinv_ref[...] * (sa_ref[...] + sb_ref[...])
                  + selfc_ref[...] * p_ref[...] + b_ref[...])


_tc_combine = pl.pallas_call(
    _cmb_body,
    grid=(_G,),
    in_specs=[
        pl.BlockSpec((_R, _H), lambda i: (i, 0)),
        pl.BlockSpec((_R, _H), lambda i: (i + _G, 0)),
        pl.BlockSpec((_R, _H), lambda i: (i, 0)),
        pl.BlockSpec((_R, 1), lambda i: (i, 0)),
        pl.BlockSpec((_R, 1), lambda i: (i, 0)),
        pl.BlockSpec((1, _H), lambda i: (0, 0)),
    ],
    out_specs=pl.BlockSpec((_R, _H), lambda i: (i, 0)),
    out_shape=jax.ShapeDtypeStruct((_N, _H), _f32),
)


def _elu(v):
    return jnp.where(v > 0, v, jnp.exp(v) - 1.0)


def _gin_body(h3_ref, sa_ref, sb_ref,
              wp1_ref, bp1_ref, wp2_ref, bp2_ref,
              wv1_ref, bv1_ref, wv2_ref, bv2_ref,
              proba_ref, value_ref, accv_ref):
    i = pl.program_id(0)
    g = h3_ref[...] + sa_ref[...] + sb_ref[...]
    tp = _elu(jnp.dot(g, wp1_ref[...], preferred_element_type=_f32)
              + bp1_ref[...])
    proba_ref[...] = (jnp.dot(tp, wp2_ref[...], preferred_element_type=_f32)
                      + bp2_ref[...])
    tv = _elu(jnp.dot(g, wv1_ref[...], preferred_element_type=_f32)
              + bv1_ref[...])
    sv = jnp.sum(tv, axis=0, keepdims=True)

    @pl.when(i == 0)
    def _():
        accv_ref[...] = sv

    @pl.when(i > 0)
    def _():
        accv_ref[...] += sv

    @pl.when(i == pl.num_programs(0) - 1)
    def _():
        value_ref[...] = (jnp.dot(accv_ref[...] / _N, wv2_ref[...],
                                  preferred_element_type=_f32)
                          + bv2_ref[...])


_tc_gin = pl.pallas_call(
    _gin_body,
    grid=(_G,),
    in_specs=[
        pl.BlockSpec((_R, _H), lambda i: (i, 0)),
        pl.BlockSpec((_R, _H), lambda i: (i, 0)),
        pl.BlockSpec((_R, _H), lambda i: (i + _G, 0)),
        pl.BlockSpec((_H, _H), lambda i: (0, 0)),
        pl.BlockSpec((1, _H), lambda i: (0, 0)),
        pl.BlockSpec((_H, 1), lambda i: (0, 0)),
        pl.BlockSpec((1, 1), lambda i: (0, 0)),
        pl.BlockSpec((_H, _H), lambda i: (0, 0)),
        pl.BlockSpec((1, _H), lambda i: (0, 0)),
        pl.BlockSpec((_H, 1), lambda i: (0, 0)),
        pl.BlockSpec((1, 1), lambda i: (0, 0)),
    ],
    out_specs=[
        pl.BlockSpec((_R, 1), lambda i: (i, 0)),
        pl.BlockSpec((1, 1), lambda i: (0, 0)),
    ],
    out_shape=[
        jax.ShapeDtypeStruct((_N, 1), _f32),
        jax.ShapeDtypeStruct((1, 1), _f32),
    ],
    scratch_shapes=[pltpu.VMEM((1, _H), _f32)],
)


def kernel(x, edge_index, W1, b1, W2, b2, W3, b3,
           Wv1, bv1, Wv2, bv2, Wp1, bp1, Wp2, bp2):
    src = edge_index[0]
    dst = edge_index[1]
    nch = _NW * _CPT
    esd = jnp.concatenate(
        [src.reshape(nch, 1, _B2), dst.reshape(nch, 1, _B2),
         jnp.zeros((nch, 6, _B2), jnp.int32)], axis=1).reshape(nch * 8, _B2)
    z128 = jnp.zeros((_N, _H), _f32)
    z1 = jnp.zeros((_N,), _f32)
    ones1 = jnp.ones((_B,), _f32)
    b1r = b1.reshape(1, _H)
    b2r = b2.reshape(1, _H)
    b3r = b3.reshape(1, _H)

    dg = _sc_deg(dst, z1, ones1).reshape(2 * _N, 1)
    p1 = _tc_matmul(x, W1)
    u1, dinv, selfc = _tc_scale(dg, dg, p1)
    s1 = _sc_seg(u1, esd, z128)
    p2, u2 = _tc_combine_mm(s1, s1, p1, dinv, selfc, b1r, W2)
    s2 = _sc_seg(u2, esd, z128)
    p3, u3 = _tc_combine_mm(s2, s2, p2, dinv, selfc, b2r, W3)
    s3 = _sc_seg(u3, esd, z128)
    h3 = _tc_combine(s3, s3, p3, dinv, selfc, b3r)
    s4 = _sc_seg(h3, esd, z128)
    proba, value = _tc_gin(h3, s4, s4, Wp1, bp1.reshape(1, _H), Wp2,
                           bp2.reshape(1, 1), Wv1, bv1.reshape(1, _H), Wv2,
                           bv2.reshape(1, 1))
    return proba, value
